# baseline (speedup 1.0000x reference)
import functools

import jax
import jax.numpy as jnp
from jax import lax
from jax.experimental import pallas as pl
from jax.experimental.pallas import tpu as pltpu
from jax.experimental.pallas import tpu_sc as plsc

_M = 256
_Q = 100
_L = 16
_NCH = _M // _L
_QCH = 7


def _tc_body(pts_ref, it_ref, st_ref, score_ref, cf_ref, tf_ref):
    f32 = jnp.float32
    pts = pts_ref[...]
    a1 = pts[:, 0:3]
    a2 = pts[:, 3:6]

    row = lax.broadcasted_iota(jnp.int32, (_M, _M), 0)
    col = lax.broadcasted_iota(jnp.int32, (_M, _M), 1)
    eye = (row == col).astype(f32)

    def gram(a):
        ab = a.astype(jnp.bfloat16)
        return lax.dot_general(ab, ab, (((1,), (1,)), ((), ())),
                               preferred_element_type=f32)

    def col2row(v):
        return jnp.sum(eye * v, axis=0, keepdims=True)

    def dist(a, G):
        n_col = jnp.sum(a * a, axis=1, keepdims=True)
        n_row = col2row(n_col)
        sq = n_col + n_row - 2.0 * G
        return jnp.sqrt(jnp.maximum(sq, 0.0) + 1e-12)

    d1 = dist(a1, gram(a1))
    d2 = dist(a2, gram(a2))
    dmat = jnp.abs(d1 - d2)

    it = it_ref[0, 0]
    st = st_ref[0, 0]
    sc = jnp.exp(-(dmat * dmat) / (2.0 * it * it))
    sc = jnp.where(sc < st, 0.0, sc)
    scb = sc.astype(jnp.bfloat16)
    sq2 = lax.dot_general(scb, scb, (((1,), (0,)), ((), ())),
                          preferred_element_type=f32)
    s = sc * sq2
    score_ref[...] = s

    snd = s * (1.0 - eye)
    degree = jnp.sum((snd != 0.0).astype(f32), axis=1, keepdims=True)

    c = jnp.where(snd > 0.0,
                  jnp.exp(jnp.log(jnp.where(snd > 0.0, snd, 1.0)) / 3.0),
                  0.0)
    cc = lax.dot_general(c, c, (((1,), (0,)), ((), ())),
                         preferred_element_type=f32,
                         precision=lax.Precision.HIGHEST)
    wijk = 0.5 * jnp.sum(cc * c, axis=1, keepdims=True)

    invalid = degree <= 1.0
    deg = jnp.where(invalid, 0.0, degree)
    f1 = jnp.where(invalid, 0.0, wijk)
    f2 = deg * (deg - 1.0) * 0.5
    sum_fenzi = jnp.sum(f1)
    sum_fenmu = jnp.sum(f2) + 1e-10
    f2 = jnp.where(invalid, 1.0, f2)
    cf = f1 / f2
    cf_ref[...] = cf
    tf_ref[...] = (sum_fenzi / sum_fenmu).reshape(1, 1)


@functools.partial(
    pl.kernel,
    out_type=jax.ShapeDtypeStruct((_L,), jnp.float32),
    mesh=plsc.VectorSubcoreMesh(core_axis_name="c", subcore_axis_name="s"),
    scratch_types=[
        pltpu.VMEM((_M,), jnp.float32),
        pltpu.VMEM((_QCH * _L,), jnp.float32),
        pltpu.VMEM((_QCH * _L,), jnp.float32),
        pltpu.VMEM((_L,), jnp.float32),
    ],
    compiler_params=pltpu.CompilerParams(needs_layout_passes=False),
)
def _otsu_sc(cf_hbm, out_hbm, cf_v, hist_v, shist_v, out_v):
    f32 = jnp.float32
    wid = lax.axis_index("c") * 16 + lax.axis_index("s")

    def sdiv(a, b):
        return jnp.max(jnp.full((_L,), a, f32) / jnp.full((_L,), b, f32))

    @pl.when(wid == 0)
    def _():
        pltpu.sync_copy(cf_hbm, cf_v)
        zeros = jnp.zeros((_L,), f32)
        for b in range(_QCH):
            hist_v[pl.ds(b * _L, _L)] = zeros
            shist_v[pl.ds(b * _L, _L)] = zeros

        maxv = f32(-jnp.inf)
        minv = f32(jnp.inf)
        total = f32(0.0)
        chunks = []
        for i in range(_NCH):
            v = cf_v[pl.ds(i * _L, _L)]
            chunks.append(v)
            maxv = jnp.maximum(maxv, jnp.max(v))
            minv = jnp.minimum(minv, jnp.min(v))
            total = total + jnp.sum(v)
        step = sdiv(maxv - minv, f32(_Q))
        stepg = jnp.where(step == 0.0, 1.0, step)

        ones = jnp.ones((_L,), f32)
        for i in range(_NCH):
            v = chunks[i]
            ids = (v / stepg).astype(jnp.int32)
            ids = jnp.where(ids >= _Q, _Q - 1, ids)
            valid = ids >= 0
            idc = jnp.where(valid, ids, 0)
            plsc.addupdate_scatter(hist_v, [idc], ones, mask=valid)
            plsc.addupdate_scatter(shist_v, [idc], v, mask=valid)

        lane = lax.iota(jnp.int32, _L)
        carry_n = f32(0.0)
        carry_f = f32(0.0)
        best_val = f32(-jnp.inf)
        best_idx = jnp.int32(0)
        for b in range(_QCH):
            h = hist_v[pl.ds(b * _L, _L)]
            sh = shist_v[pl.ds(b * _L, _L)]
            n1 = plsc.cumsum(h) + carry_n
            fore = plsc.cumsum(sh) + carry_f
            carry_n = jnp.max(n1)
            carry_f = jnp.max(fore)
            n2 = f32(_M) - n1
            m1 = fore / jnp.where(n1 == 0.0, 1.0, n1)
            m2 = (total - fore) / jnp.where(n2 == 0.0, 1.0, n2)
            d = m1 - m2
            sb = n1 * n2 * d * d
            qok = (n1 > 0.0) & (n2 > 0.0) & (lane + b * _L < _Q)
            sb = jnp.where(qok, sb, -jnp.inf)
            mc = jnp.max(sb)
            pos = jnp.min(jnp.where(sb == mc, lane, _L))
            new = mc > best_val
            best_val = jnp.where(new, mc, best_val)
            best_idx = jnp.where(new, pos + b * _L, best_idx)
        tval = sdiv(best_idx.astype(f32) * (maxv - minv), f32(_Q))
        th = jnp.where(best_val > -1000.0, tval, 0.0)
        out_v[...] = jnp.zeros((_L,), f32) + th
        pltpu.sync_copy(out_v, out_hbm)


def kernel(points, inlier_thre, score_thresh):
    pts = points.reshape(_M, 6)
    it = jnp.asarray(inlier_thre, jnp.float32).reshape(1, 1)
    st = jnp.asarray(score_thresh, jnp.float32).reshape(1, 1)
    out_shapes = (
        jax.ShapeDtypeStruct((_M, _M), jnp.float32),
        jax.ShapeDtypeStruct((_M, 1), jnp.float32),
        jax.ShapeDtypeStruct((1, 1), jnp.float32),
    )
    s, cf, tf = pl.pallas_call(
        _tc_body,
        out_shape=out_shapes,
    )(pts, it, st)
    th = _otsu_sc(cf.reshape(_M))[0]
    return (s.reshape(1, _M, _M), cf.reshape(_M),
            tf.reshape(()), th.reshape(()))


# speedup vs baseline: 1.0630x; 1.0630x over previous
import functools

import jax
import jax.numpy as jnp
from jax import lax
from jax.experimental import pallas as pl
from jax.experimental.pallas import tpu as pltpu
from jax.experimental.pallas import tpu_sc as plsc

_M = 256
_Q = 100
_L = 16
_NCH = _M // _L
_QCH = 7


def _tc_body(pts_ref, it_ref, st_ref, score_ref, cf_ref, tf_ref):
    f32 = jnp.float32
    pts = pts_ref[...]
    a1 = pts[:, 0:3]
    a2 = pts[:, 3:6]

    row = lax.broadcasted_iota(jnp.int32, (_M, _M), 0)
    col = lax.broadcasted_iota(jnp.int32, (_M, _M), 1)
    eye = (row == col).astype(f32)

    def gram(a):
        ab = a.astype(jnp.bfloat16)
        return lax.dot_general(ab, ab, (((1,), (1,)), ((), ())),
                               preferred_element_type=f32)

    def col2row(v):
        return jnp.sum(eye * v, axis=0, keepdims=True)

    def dist(a, G):
        n_col = jnp.sum(a * a, axis=1, keepdims=True)
        n_row = col2row(n_col)
        sq = n_col + n_row - 2.0 * G
        return jnp.sqrt(jnp.maximum(sq, 0.0) + 1e-12)

    d1 = dist(a1, gram(a1))
    d2 = dist(a2, gram(a2))
    dmat = jnp.abs(d1 - d2)

    it = it_ref[0, 0]
    st = st_ref[0, 0]
    sc = jnp.exp(-(dmat * dmat) / (2.0 * it * it))
    sc = jnp.where(sc < st, 0.0, sc)
    scb = sc.astype(jnp.bfloat16)
    sq2 = lax.dot_general(scb, scb, (((1,), (0,)), ((), ())),
                          preferred_element_type=f32)
    s = sc * sq2
    score_ref[...] = s

    snd = s * (1.0 - eye)
    degree = jnp.sum((snd != 0.0).astype(f32), axis=1, keepdims=True)

    c = jnp.where(snd > 0.0,
                  jnp.exp(jnp.log(jnp.where(snd > 0.0, snd, 1.0)) / 3.0),
                  0.0)
    cc = lax.dot_general(c, c, (((1,), (0,)), ((), ())),
                         preferred_element_type=f32,
                         precision=lax.Precision.HIGHEST)
    wijk = 0.5 * jnp.sum(cc * c, axis=1, keepdims=True)

    invalid = degree <= 1.0
    deg = jnp.where(invalid, 0.0, degree)
    f1 = jnp.where(invalid, 0.0, wijk)
    f2 = deg * (deg - 1.0) * 0.5
    sum_fenzi = jnp.sum(f1)
    sum_fenmu = jnp.sum(f2) + 1e-10
    f2 = jnp.where(invalid, 1.0, f2)
    cf = f1 / f2
    cf_ref[...] = cf
    tf_ref[...] = (sum_fenzi / sum_fenmu).reshape(1, 1)


@functools.partial(
    pl.kernel,
    out_type=jax.ShapeDtypeStruct((_L,), jnp.float32),
    mesh=plsc.VectorSubcoreMesh(core_axis_name="c", subcore_axis_name="s",
                                num_cores=1),
    scratch_types=[
        pltpu.VMEM((_M,), jnp.float32),
        pltpu.VMEM((_QCH * _L,), jnp.float32),
        pltpu.VMEM((_QCH * _L,), jnp.float32),
        pltpu.VMEM((_L,), jnp.float32),
    ],
    compiler_params=pltpu.CompilerParams(needs_layout_passes=False),
)
def _otsu_sc(cf_hbm, out_hbm, cf_v, hist_v, shist_v, out_v):
    f32 = jnp.float32
    wid = lax.axis_index("c") * 16 + lax.axis_index("s")

    def sdiv(a, b):
        return jnp.max(jnp.full((_L,), a, f32) / jnp.full((_L,), b, f32))

    @pl.when(wid == 0)
    def _():
        pltpu.sync_copy(cf_hbm, cf_v)
        zeros = jnp.zeros((_L,), f32)
        for b in range(_QCH):
            hist_v[pl.ds(b * _L, _L)] = zeros
            shist_v[pl.ds(b * _L, _L)] = zeros

        maxv = f32(-jnp.inf)
        minv = f32(jnp.inf)
        total = f32(0.0)
        chunks = []
        for i in range(_NCH):
            v = cf_v[pl.ds(i * _L, _L)]
            chunks.append(v)
            maxv = jnp.maximum(maxv, jnp.max(v))
            minv = jnp.minimum(minv, jnp.min(v))
            total = total + jnp.sum(v)
        step = sdiv(maxv - minv, f32(_Q))
        stepg = jnp.where(step == 0.0, 1.0, step)

        ones = jnp.ones((_L,), f32)
        for i in range(_NCH):
            v = chunks[i]
            ids = (v / stepg).astype(jnp.int32)
            ids = jnp.where(ids >= _Q, _Q - 1, ids)
            valid = ids >= 0
            idc = jnp.where(valid, ids, 0)
            plsc.addupdate_scatter(hist_v, [idc], ones, mask=valid)
            plsc.addupdate_scatter(shist_v, [idc], v, mask=valid)

        lane = lax.iota(jnp.int32, _L)
        carry_n = f32(0.0)
        carry_f = f32(0.0)
        best_val = f32(-jnp.inf)
        best_idx = jnp.int32(0)
        for b in range(_QCH):
            h = hist_v[pl.ds(b * _L, _L)]
            sh = shist_v[pl.ds(b * _L, _L)]
            n1 = plsc.cumsum(h) + carry_n
            fore = plsc.cumsum(sh) + carry_f
            carry_n = jnp.max(n1)
            carry_f = jnp.max(fore)
            n2 = f32(_M) - n1
            m1 = fore / jnp.where(n1 == 0.0, 1.0, n1)
            m2 = (total - fore) / jnp.where(n2 == 0.0, 1.0, n2)
            d = m1 - m2
            sb = n1 * n2 * d * d
            qok = (n1 > 0.0) & (n2 > 0.0) & (lane + b * _L < _Q)
            sb = jnp.where(qok, sb, -jnp.inf)
            mc = jnp.max(sb)
            pos = jnp.min(jnp.where(sb == mc, lane, _L))
            new = mc > best_val
            best_val = jnp.where(new, mc, best_val)
            best_idx = jnp.where(new, pos + b * _L, best_idx)
        tval = sdiv(best_idx.astype(f32) * (maxv - minv), f32(_Q))
        th = jnp.where(best_val > -1000.0, tval, 0.0)
        out_v[...] = jnp.zeros((_L,), f32) + th
        pltpu.sync_copy(out_v, out_hbm)


def kernel(points, inlier_thre, score_thresh):
    pts = points.reshape(_M, 6)
    it = jnp.asarray(inlier_thre, jnp.float32).reshape(1, 1)
    st = jnp.asarray(score_thresh, jnp.float32).reshape(1, 1)
    out_shapes = (
        jax.ShapeDtypeStruct((_M, _M), jnp.float32),
        jax.ShapeDtypeStruct((_M, 1), jnp.float32),
        jax.ShapeDtypeStruct((1, 1), jnp.float32),
    )
    s, cf, tf = pl.pallas_call(
        _tc_body,
        out_shape=out_shapes,
    )(pts, it, st)
    th = _otsu_sc(cf.reshape(_M))[0]
    return (s.reshape(1, _M, _M), cf.reshape(_M),
            tf.reshape(()), th.reshape(()))
